# baseline (speedup 1.0000x reference)
import functools

import jax
import jax.numpy as jnp
from jax import lax
from jax.experimental import pallas as pl
from jax.experimental.pallas import tpu as pltpu
from jax.experimental.pallas import tpu_sc as plsc

ROWS_PER_BLOCK = 26
BLOCKS_PER_CHUNK = 8
CHUNK = ROWS_PER_BLOCK * BLOCKS_PER_CHUNK
NBUF = 4


@functools.cache
def _make_gather(n_b: int, n_k: int, d: int):
    info = plsc.get_sparse_core_info()
    nc, ns = info.num_cores, info.num_subcores
    nw = nc * ns
    n_chunks = (n_b // BLOCKS_PER_CHUNK)
    chunks_per_w = n_chunks // nw
    n_groups = chunks_per_w // NBUF
    mesh = plsc.VectorSubcoreMesh(core_axis_name="c", subcore_axis_name="s")

    @functools.partial(
        pl.kernel,
        mesh=mesh,
        out_type=jax.ShapeDtypeStruct((n_b, n_k, d), jnp.float32),
        scratch_types=[
            pltpu.VMEM((chunks_per_w, CHUNK), jnp.int32),
            pltpu.VMEM((NBUF, CHUNK, d), jnp.float32),
            [pltpu.SemaphoreType.DMA] * NBUF,
            [pltpu.SemaphoreType.DMA] * NBUF,
        ],
    )
    def gather_kernel(table_hbm, idx_hbm, out_hbm, idx_v, bufs, gsems, ssems):
        wid = lax.axis_index("s") * nc + lax.axis_index("c")
        base = wid * chunks_per_w
        pltpu.sync_copy(idx_hbm.at[pl.ds(base, chunks_per_w)], idx_v)

        def stage(b, j):
            del b, j

        _windows = [(i * 16, 0) for i in range(CHUNK // 16)]
        if CHUNK % 16:
            _windows.append((CHUNK - 16, 16 - CHUNK % 16))

        def gather_issue(b, j):
            for off, lo in _windows:
                vec = idx_v[j, pl.ds(off, 16)]
                for lane in range(lo, 16):
                    pltpu.async_copy(
                        table_hbm.at[vec[lane]], bufs.at[b, off + lane], gsems[b]
                    )

        def drain_gather(b):
            pltpu.make_async_copy(
                table_hbm.at[pl.ds(0, CHUNK)], bufs.at[b], gsems[b]
            ).wait()

        def store(b, j):
            c = base + j
            for i in range(BLOCKS_PER_CHUNK):
                pltpu.async_copy(
                    bufs.at[b, pl.ds(i * ROWS_PER_BLOCK, ROWS_PER_BLOCK)],
                    out_hbm.at[c * BLOCKS_PER_CHUNK + i],
                    ssems[b],
                )

        def drain_store(b, j):
            c = base + j
            for i in range(BLOCKS_PER_CHUNK):
                pltpu.make_async_copy(
                    bufs.at[b, pl.ds(i * ROWS_PER_BLOCK, ROWS_PER_BLOCK)],
                    out_hbm.at[c * BLOCKS_PER_CHUNK + i],
                    ssems[b],
                ).wait()

        for b in range(NBUF):
            stage(b, b)
            gather_issue(b, b)

        def group_body(g, carry):
            j0 = g * NBUF
            for b in range(NBUF):
                drain_gather(b)
                store(b, j0 + b)
            for b in range(NBUF):
                drain_store(b, j0 + b)
                stage(b, j0 + NBUF + b)
                gather_issue(b, j0 + NBUF + b)
            return carry

        lax.fori_loop(0, n_groups - 1, group_body, 0)

        j0 = (n_groups - 1) * NBUF
        for b in range(NBUF):
            drain_gather(b)
            store(b, j0 + b)
        for b in range(NBUF):
            drain_store(b, j0 + b)

    return gather_kernel


def kernel(vars_, index):
    n_b, n_k = index.shape
    d = vars_.shape[1]
    idx = index.reshape(n_b // BLOCKS_PER_CHUNK, CHUNK).astype(jnp.int32)
    return _make_gather(n_b, n_k, d)(vars_, idx)


# speedup vs baseline: 1.0184x; 1.0184x over previous
import functools

import jax
import jax.numpy as jnp
from jax import lax
from jax.experimental import pallas as pl
from jax.experimental.pallas import tpu as pltpu
from jax.experimental.pallas import tpu_sc as plsc

ROWS_PER_BLOCK = 26
BLOCKS_PER_CHUNK = 4
CHUNK = ROWS_PER_BLOCK * BLOCKS_PER_CHUNK
NBUF = 4


@functools.cache
def _make_gather(n_b: int, n_k: int, d: int):
    info = plsc.get_sparse_core_info()
    nc, ns = info.num_cores, info.num_subcores
    nw = nc * ns
    n_chunks = (n_b // BLOCKS_PER_CHUNK)
    chunks_per_w = n_chunks // nw
    n_groups = chunks_per_w // NBUF
    mesh = plsc.VectorSubcoreMesh(core_axis_name="c", subcore_axis_name="s")

    @functools.partial(
        pl.kernel,
        mesh=mesh,
        out_type=jax.ShapeDtypeStruct((n_b, n_k, d), jnp.float32),
        scratch_types=[
            pltpu.VMEM((chunks_per_w, CHUNK), jnp.int32),
            pltpu.VMEM((NBUF, CHUNK, d), jnp.float32),
            [pltpu.SemaphoreType.DMA] * NBUF,
            [pltpu.SemaphoreType.DMA] * NBUF,
        ],
    )
    def gather_kernel(table_hbm, idx_hbm, out_hbm, idx_v, bufs, gsems, ssems):
        wid = lax.axis_index("s") * nc + lax.axis_index("c")
        base = wid * chunks_per_w
        pltpu.sync_copy(idx_hbm.at[pl.ds(base, chunks_per_w)], idx_v)

        _windows = [(i * 16, 0) for i in range(CHUNK // 16)]
        if CHUNK % 16:
            _windows.append((CHUNK - 16, 16 - CHUNK % 16))

        def gather_issue(b, j):
            for off, lo in _windows:
                vec = idx_v[j, pl.ds(off, 16)]
                for lane in range(lo, 16):
                    pltpu.async_copy(
                        table_hbm.at[vec[lane]], bufs.at[b, off + lane], gsems[b]
                    )

        def drain_gather(b):
            pltpu.make_async_copy(
                table_hbm.at[pl.ds(0, CHUNK)], bufs.at[b], gsems[b]
            ).wait()

        def store(b, j):
            c = base + j
            for i in range(BLOCKS_PER_CHUNK):
                pltpu.async_copy(
                    bufs.at[b, pl.ds(i * ROWS_PER_BLOCK, ROWS_PER_BLOCK)],
                    out_hbm.at[c * BLOCKS_PER_CHUNK + i],
                    ssems[b],
                )

        def drain_store(b, j):
            c = base + j
            for i in range(BLOCKS_PER_CHUNK):
                pltpu.make_async_copy(
                    bufs.at[b, pl.ds(i * ROWS_PER_BLOCK, ROWS_PER_BLOCK)],
                    out_hbm.at[c * BLOCKS_PER_CHUNK + i],
                    ssems[b],
                ).wait()

        for b in range(NBUF):
            gather_issue(b, b)

        def group_body(g, carry):
            j0 = g * NBUF
            for b in range(NBUF):
                drain_gather(b)
                store(b, j0 + b)
            for b in range(NBUF):
                drain_store(b, j0 + b)
                gather_issue(b, j0 + NBUF + b)
            return carry

        lax.fori_loop(0, n_groups - 1, group_body, 0)

        j0 = (n_groups - 1) * NBUF
        for b in range(NBUF):
            drain_gather(b)
            store(b, j0 + b)
        for b in range(NBUF):
            drain_store(b, j0 + b)

    return gather_kernel


def kernel(vars_, index):
    n_b, n_k = index.shape
    d = vars_.shape[1]
    idx = index.reshape(n_b // BLOCKS_PER_CHUNK, CHUNK).astype(jnp.int32)
    return _make_gather(n_b, n_k, d)(vars_, idx)
